# baseline (speedup 1.0000x reference)
import functools

import jax
import jax.numpy as jnp
from jax import lax
from jax.experimental import pallas as pl
from jax.experimental.pallas import tpu as pltpu
from jax.experimental.pallas import tpu_sc as plsc

N = 10000
E = 320000
D = 128
H = 8
C1 = 8
NCLS = 16
HC1 = H * C1
ROWW = 128

K = 64
EPAD = 327680
EW1 = EPAD // 32
EW2 = EPAD // 16
NPAD = 10240
RPS = NPAD // 16
DUMP = N

_BLK = 1000
_GRID = N // _BLK


def _self_attn(hw, bdl_ref, bdr_ref, ob_ref, dup_ref, de_ref):
    al = jnp.dot(hw, bdl_ref[...], preferred_element_type=jnp.float32)
    ar = jnp.dot(hw, bdr_ref[...], preferred_element_type=jnp.float32)
    lg = jnp.dot(hw * hw, ob_ref[...], preferred_element_type=jnp.float32)
    alpha = (al + ar) * (1.0 / (1.0 + jnp.exp(-lg)))
    alpha = jnp.where(alpha >= 0.0, alpha, 0.2 * alpha)
    a_s = jnp.exp(alpha)
    aldup = jnp.dot(al, dup_ref[...], preferred_element_type=jnp.float32)
    ardup = jnp.dot(ar, dup_ref[...], preferred_element_type=jnp.float32)
    selfy = hw * jnp.dot(a_s, de_ref[...], preferred_element_type=jnp.float32)
    selfd = jnp.dot(a_s, dup_ref[...], preferred_element_type=jnp.float32)
    return aldup, ardup, selfy, selfd


def _pack_rows(hw, aldup, ardup):
    z = jnp.zeros((hw.shape[0], 32), jnp.float32)
    return jnp.concatenate([hw, aldup, ardup, z], axis=1)


def _pack(hext_ref, self_ref, hw, aldup, ardup, selfy, selfd):
    hext_ref[...] = _pack_rows(hw, aldup, ardup)
    self_ref[...] = jnp.concatenate([selfy, selfd], axis=1)


def _prep1_body(x_ref, w_ref, bdl_ref, bdr_ref, ob_ref, dup_ref, de_ref,
                hext_ref, self_ref):
    hw = jnp.dot(x_ref[...], w_ref[...], preferred_element_type=jnp.float32)
    aldup, ardup, selfy, selfd = _self_attn(hw, bdl_ref, bdr_ref, ob_ref,
                                            dup_ref, de_ref)
    _pack(hext_ref, self_ref, hw, aldup, ardup, selfy, selfd)


def _mid_body(p0_ref, p1_ref, s1_ref, b1_ref, wa_ref, wb_ref,
              bdla_ref, bdra_ref, bdlb_ref, bdrb_ref, ob_ref,
              dup_ref, de4_ref, de8_ref, hext2_ref,
              selfa_ref, selfb_ref):
    t = p0_ref[:, 0:80] + p1_ref[:, 0:80] + s1_ref[...]
    den = jnp.dot(t[:, 64:72], de8_ref[...], preferred_element_type=jnp.float32)
    pre = t[:, 0:64] / den + b1_ref[...]
    h1 = jnp.where(pre > 0.0, pre, jnp.exp(jnp.minimum(pre, 0.0)) - 1.0)
    hwa = jnp.dot(h1, wa_ref[...], preferred_element_type=jnp.float32)
    hwb = jnp.dot(h1, wb_ref[...], preferred_element_type=jnp.float32)
    ala, ara, sya, sda = _self_attn(hwa, bdla_ref, bdra_ref, ob_ref,
                                    dup_ref, de4_ref)
    alb, arb, syb, sdb = _self_attn(hwb, bdlb_ref, bdrb_ref, ob_ref,
                                    dup_ref, de4_ref)
    hext2_ref[:, 0:ROWW] = _pack_rows(hwa, ala, ara)
    hext2_ref[:, ROWW:2 * ROWW] = _pack_rows(hwb, alb, arb)
    selfa_ref[...] = jnp.concatenate([sya, sda], axis=1)
    selfb_ref[...] = jnp.concatenate([syb, sdb], axis=1)


def _fin_body(pa_ref, pb_ref, sa_ref, sb_ref, b2_ref, de4_ref, mean_ref,
              out_ref):
    ta = pa_ref[:, 0:80] + sa_ref[...]
    tb = pb_ref[:, 0:80] + sb_ref[...]
    dena = jnp.dot(ta[:, 64:68], de4_ref[...], preferred_element_type=jnp.float32)
    denb = jnp.dot(tb[:, 64:68], de4_ref[...], preferred_element_type=jnp.float32)
    u = ta[:, 0:64] / dena + tb[:, 0:64] / denb
    o = jnp.dot(u, mean_ref[...], preferred_element_type=jnp.float32) + b2_ref[...]
    m = jnp.max(o, axis=-1, keepdims=True)
    z = o - m
    lse = jnp.log(jnp.sum(jnp.exp(z), axis=-1, keepdims=True))
    out_ref[...] = z - lse


def _row_spec(w):
    return pl.BlockSpec((_BLK, w), lambda i: (i, 0))


def _full_spec(r, c):
    return pl.BlockSpec((r, c), lambda i: (0, 0))


def _edge_body(table, srce, dste, out0, out1, accum,
               srcv0, dstv0, srcv1, dstv1, drv0, drv1,
               gbufs, xj0, xi0, xj1, xi1, semI0, semI1, semG0, semG1,
               *, nfold, interleaved):
    f32 = jnp.float32
    cid = lax.axis_index("c")
    sid = lax.axis_index("s")
    lanes = lax.broadcasted_iota(jnp.int32, (16,), 0)
    rot8 = (lanes + 8) % 16
    rot4 = (lanes + 4) % 16

    if interleaved:
        ew = EW2
        base0 = sid * ew
    else:
        ew = EW1
        base0 = (sid * 2 + cid) * ew
    nch = ew // K
    npairs = nch // 2

    def zrow(i, carry):
        for j in range(ROWW // 16):
            xj0[i, pl.ds(j * 16, 16)] = jnp.zeros((16,), f32)
        return carry
    lax.fori_loop(0, K, zrow, 0)
    rb = sid * RPS
    for j in range(RPS // K):
        pltpu.async_copy(xj0, accum.at[pl.ds(rb + j * K, K)], semI0)

    def idx_fetch(c, sv, dv, sem):
        base = base0 + c * K
        pltpu.async_copy(srce.at[pl.ds(base, K)], sv, sem)
        pltpu.async_copy(dste.at[pl.ds(base, K)], dv, sem)

    def idx_wait(sv, dv, sem):
        pltpu.make_async_copy(srce.at[pl.ds(0, K)], sv, sem).wait()
        pltpu.make_async_copy(dste.at[pl.ds(0, K)], dv, sem).wait()

    def gath(sv, dv, xj_, xi_, sem):
        pltpu.async_copy(table.at[sv], xj_, sem)
        pltpu.async_copy(table.at[dv], xi_, sem)

    def gath_wait(xj_, xi_, sem):
        pltpu.make_async_copy(table.at[pl.ds(0, K)], xj_, sem).wait()
        pltpu.make_async_copy(table.at[pl.ds(0, K)], xi_, sem).wait()

    def sel(sv, dv, slot):
        if not interleaved:
            return sv, dv
        svg, dvg = gbufs[slot]
        for j in range(K // 16):
            s16 = sv[pl.ds(j * 16, 16)]
            d16 = dv[pl.ds(j * 16, 16)]
            svg[pl.ds(j * 16, 16)] = s16 + s16 + cid
            dvg[pl.ds(j * 16, 16)] = d16 + d16 + cid
        return svg, dvg

    def mk_drv(sv, dv, drv_):
        for j in range(K // 16):
            s16 = sv[pl.ds(j * 16, 16)]
            d16 = dv[pl.ds(j * 16, 16)]
            drv_[pl.ds(j * 16, 16)] = jnp.where(s16 == d16, jnp.int32(DUMP),
                                                d16)

    def edge_one(xj, xi, e):
        vj = [xj[e, pl.ds(16 * k, 16)] for k in range(4)]
        vi = [xi[e, pl.ds(16 * k, 16)] for k in range(4)]
        p = vj[0] * vi[0]
        for k in range(1, 4):
            p = p + vj[k] * vi[k]
        p = p + p[rot8]
        if nfold == 2:
            p = p + p[rot4]
        s = xj[e, pl.ds(64, 16)] + xi[e, pl.ds(80, 16)]
        alpha = s * (1.0 / (1.0 + jnp.exp(-p)))
        alpha = jnp.maximum(alpha, 0.2 * alpha)
        a = jnp.exp(alpha)
        for k in range(4):
            xj[e, pl.ds(16 * k, 16)] = vj[k] * a
        xj[e, pl.ds(64, 16)] = a

    def edges(xj, xi):
        def e2(i, c):
            edge_one(xj, xi, 2 * i)
            edge_one(xj, xi, 2 * i + 1)
            return c
        lax.fori_loop(0, K // 2, e2, 0)

    pltpu.sync_copy(srce.at[pl.ds(base0, K)], srcv0)
    pltpu.sync_copy(dste.at[pl.ds(base0, K)], dstv0)
    g0s, g0d = sel(srcv0, dstv0, 0)
    idx_fetch(1, srcv1, dstv1, semI1)
    for j in range(RPS // K):
        pltpu.make_async_copy(xj0, accum.at[pl.ds(rb, K)], semI0).wait()
    gath(g0s, g0d, xj0, xi0, semG0)
    plsc.subcore_barrier()

    def pair(p, carry):
        c0 = 2 * p
        more = p < npairs - 1

        idx_wait(srcv1, dstv1, semI1)
        g1s, g1d = sel(srcv1, dstv1, 1)
        gath(g1s, g1d, xj1, xi1, semG1)
        mk_drv(srcv0, dstv0, drv0)
        gath_wait(xj0, xi0, semG0)

        @pl.when(more)
        def _():
            idx_fetch(c0 + 2, srcv0, dstv0, semI0)
        edges(xj0, xi0)
        pltpu.sync_copy(xj0, accum.at[drv0], add=True)

        @pl.when(more)
        def _():
            idx_wait(srcv0, dstv0, semI0)
            gs, gd = sel(srcv0, dstv0, 0)
            gath(gs, gd, xj0, xi0, semG0)
        mk_drv(srcv1, dstv1, drv1)
        gath_wait(xj1, xi1, semG1)

        @pl.when(more)
        def _():
            idx_fetch(c0 + 3, srcv1, dstv1, semI1)
        edges(xj1, xi1)
        pltpu.sync_copy(xj1, accum.at[drv1], add=True)
        return carry
    lax.fori_loop(0, npairs, pair, 0)
    plsc.subcore_barrier()

    @pl.when(cid == 0)
    def _():
        pltpu.sync_copy(accum.at[pl.ds(rb, RPS)], out0.at[pl.ds(rb, RPS)])

    @pl.when(cid == 1)
    def _():
        pltpu.sync_copy(accum.at[pl.ds(rb, RPS)], out1.at[pl.ds(rb, RPS)])


def _make_edge_kernel(interleaved, nfold):
    mesh = plsc.VectorSubcoreMesh(core_axis_name="c", subcore_axis_name="s",
                                  num_cores=2, num_subcores=16)
    f32 = jnp.float32

    def body(table, srce, dste, out0, out1, accum,
             srcv0, dstv0, srcv1, dstv1, drv0, drv1, *rest):
        if interleaved:
            sg0, dg0, sg1, dg1 = rest[:4]
            gbufs = ((sg0, dg0), (sg1, dg1))
            rest = rest[4:]
        else:
            gbufs = None
        (xj0, xi0, xj1, xi1, semI0, semI1, semG0, semG1) = rest
        _edge_body(table, srce, dste, out0, out1, accum,
                   srcv0, dstv0, srcv1, dstv1, drv0, drv1,
                   gbufs, xj0, xi0, xj1, xi1, semI0, semI1, semG0, semG1,
                   nfold=nfold, interleaved=interleaved)

    nivec = 10 if interleaved else 6
    scratch = ([pltpu.VMEM_SHARED((NPAD, ROWW), f32)]
               + [pltpu.VMEM((K,), jnp.int32) for _ in range(nivec)]
               + [pltpu.VMEM((K, ROWW), f32) for _ in range(4)]
               + [pltpu.SemaphoreType.DMA for _ in range(4)])

    return functools.partial(
        pl.kernel,
        out_type=[jax.ShapeDtypeStruct((NPAD, ROWW), f32),
                  jax.ShapeDtypeStruct((NPAD, ROWW), f32)],
        mesh=mesh,
        scratch_types=scratch,
    )(body)


_edge_kernel_cache = {}


def _edge_kernel_for(interleaved, nfold):
    kk = (interleaved, nfold)
    if kk not in _edge_kernel_cache:
        _edge_kernel_cache[kk] = _make_edge_kernel(interleaved, nfold)
    return _edge_kernel_cache[kk]


def _cm_blockdiag(att2, nh):
    ch = att2.shape[1]
    return (att2.T.reshape(ch, nh, 1)
            * jnp.eye(nh, dtype=jnp.float32).reshape(1, nh, nh)).reshape(ch * nh, nh)


def kernel(x, edge_index, W1, att_l1, att_r1, b1, W2, att_l2, att_r2, b2):
    f32 = jnp.float32
    i32 = jnp.int32
    src = edge_index[0]
    dst = edge_index[1]

    zpad = jnp.zeros((EPAD - E,), i32)
    srcp = jnp.concatenate([src, zpad])
    dstp = jnp.concatenate([dst, zpad])

    cm1 = (jnp.arange(HC1, dtype=i32) % H) * C1 + jnp.arange(HC1, dtype=i32) // H
    w1cm = W1[:, cm1]
    at_l1 = att_l1.reshape(H, C1)
    at_r1 = att_r1.reshape(H, C1)
    bdl1 = _cm_blockdiag(at_l1, H)
    bdr1 = _cm_blockdiag(at_r1, H)
    ob1 = jnp.kron(jnp.ones((C1, 1), f32), jnp.eye(H, dtype=f32))
    dup8 = jnp.kron(jnp.ones((1, 2), f32), jnp.eye(H, dtype=f32))
    de8 = jnp.kron(jnp.ones((1, C1), f32), jnp.eye(H, dtype=f32))

    cmh = (jnp.arange(64, dtype=i32) % 4) * NCLS + jnp.arange(64, dtype=i32) // 4
    w2r = W2[cm1, :]
    w2a = w2r[:, cmh]
    w2b = w2r[:, cmh + 4 * NCLS]
    at_l2 = att_l2.reshape(H, NCLS)
    at_r2 = att_r2.reshape(H, NCLS)
    bdl2a = _cm_blockdiag(at_l2[0:4], 4)
    bdr2a = _cm_blockdiag(at_r2[0:4], 4)
    bdl2b = _cm_blockdiag(at_l2[4:8], 4)
    bdr2b = _cm_blockdiag(at_r2[4:8], 4)
    ob2 = jnp.kron(jnp.ones((NCLS, 1), f32), jnp.eye(4, dtype=f32))
    dup4 = jnp.kron(jnp.ones((1, 4), f32), jnp.eye(4, dtype=f32))
    de4 = jnp.kron(jnp.ones((1, NCLS), f32), jnp.eye(4, dtype=f32))
    meancm = jnp.kron(jnp.eye(NCLS, dtype=f32), jnp.full((4, 1), 1.0 / H, f32))

    hext1, self1 = pl.pallas_call(
        _prep1_body,
        grid=(_GRID,),
        in_specs=[_row_spec(D), _full_spec(D, HC1), _full_spec(HC1, H),
                  _full_spec(HC1, H), _full_spec(HC1, H), _full_spec(H, 16),
                  _full_spec(H, HC1)],
        out_specs=[_row_spec(ROWW), _row_spec(80)],
        out_shape=[jax.ShapeDtypeStruct((N, ROWW), f32),
                   jax.ShapeDtypeStruct((N, 80), f32)],
    )(x, w1cm, bdl1, bdr1, ob1, dup8, de8)

    p10, p11 = _edge_kernel_for(False, 1)(hext1, srcp, dstp)

    hext2, selfa, selfb = pl.pallas_call(
        _mid_body,
        grid=(_GRID,),
        in_specs=[_row_spec(ROWW), _row_spec(ROWW), _row_spec(80),
                  _full_spec(1, HC1), _full_spec(HC1, 64), _full_spec(HC1, 64),
                  _full_spec(64, 4), _full_spec(64, 4), _full_spec(64, 4),
                  _full_spec(64, 4), _full_spec(64, 4), _full_spec(4, 16),
                  _full_spec(4, 64), _full_spec(H, HC1)],
        out_specs=[_row_spec(2 * ROWW), _row_spec(80), _row_spec(80)],
        out_shape=[jax.ShapeDtypeStruct((N, 2 * ROWW), f32),
                   jax.ShapeDtypeStruct((N, 80), f32),
                   jax.ShapeDtypeStruct((N, 80), f32)],
    )(p10, p11, self1, b1[cm1].reshape(1, HC1), w2a, w2b,
      bdl2a, bdr2a, bdl2b, bdr2b, ob2, dup4, de4, de8)

    p20, p21 = _edge_kernel_for(True, 2)(hext2.reshape(2 * N, ROWW),
                                         srcp, dstp)

    out = pl.pallas_call(
        _fin_body,
        grid=(_GRID,),
        in_specs=[_row_spec(ROWW), _row_spec(ROWW), _row_spec(80),
                  _row_spec(80), _full_spec(1, NCLS), _full_spec(4, 64),
                  _full_spec(64, NCLS)],
        out_specs=[_row_spec(NCLS)],
        out_shape=[jax.ShapeDtypeStruct((N, NCLS), f32)],
    )(p20, p21, selfa, selfb, b2.reshape(1, NCLS), de4, meancm)[0]

    return out, jnp.float32(0.0)


# speedup vs baseline: 1.0441x; 1.0441x over previous
import functools

import jax
import jax.numpy as jnp
from jax import lax
from jax.experimental import pallas as pl
from jax.experimental.pallas import tpu as pltpu
from jax.experimental.pallas import tpu_sc as plsc

N = 10000
E = 320000
D = 128
H = 8
C1 = 8
NCLS = 16
HC1 = H * C1
ROWW = 128

K = 64
EPAD = 327680
EW1 = EPAD // 32
EW2 = EPAD // 16
NPAD = 10240
RPS = NPAD // 16
DUMP = N

_BLK = 1000
_GRID = N // _BLK


def _self_attn(hw, bdl_ref, bdr_ref, ob_ref, dup_ref, de_ref):
    al = jnp.dot(hw, bdl_ref[...], preferred_element_type=jnp.float32)
    ar = jnp.dot(hw, bdr_ref[...], preferred_element_type=jnp.float32)
    lg = jnp.dot(hw * hw, ob_ref[...], preferred_element_type=jnp.float32)
    alpha = (al + ar) * (1.0 / (1.0 + jnp.exp(-lg)))
    alpha = jnp.where(alpha >= 0.0, alpha, 0.2 * alpha)
    a_s = jnp.exp(alpha)
    aldup = jnp.dot(al, dup_ref[...], preferred_element_type=jnp.float32)
    ardup = jnp.dot(ar, dup_ref[...], preferred_element_type=jnp.float32)
    selfy = hw * jnp.dot(a_s, de_ref[...], preferred_element_type=jnp.float32)
    selfd = jnp.dot(a_s, dup_ref[...], preferred_element_type=jnp.float32)
    return aldup, ardup, selfy, selfd


def _pack_rows(hw, aldup, ardup):
    z = jnp.zeros((hw.shape[0], 32), jnp.float32)
    return jnp.concatenate([hw, aldup, ardup, z], axis=1)


def _pack(hext_ref, self_ref, hw, aldup, ardup, selfy, selfd):
    hext_ref[...] = _pack_rows(hw, aldup, ardup)
    self_ref[...] = jnp.concatenate([selfy, selfd], axis=1)


def _prep1_body(x_ref, w_ref, bdl_ref, bdr_ref, ob_ref, dup_ref, de_ref,
                hext_ref, self_ref):
    hw = jnp.dot(x_ref[...], w_ref[...], preferred_element_type=jnp.float32)
    aldup, ardup, selfy, selfd = _self_attn(hw, bdl_ref, bdr_ref, ob_ref,
                                            dup_ref, de_ref)
    _pack(hext_ref, self_ref, hw, aldup, ardup, selfy, selfd)


def _mid_body(p0_ref, p1_ref, s1_ref, b1_ref, wa_ref, wb_ref,
              bdla_ref, bdra_ref, bdlb_ref, bdrb_ref, ob_ref,
              dup_ref, de4_ref, de8_ref, hext2_ref,
              selfa_ref, selfb_ref):
    t = p0_ref[:, 0:80] + p1_ref[:, 0:80] + s1_ref[...]
    den = jnp.dot(t[:, 64:72], de8_ref[...], preferred_element_type=jnp.float32)
    pre = t[:, 0:64] / den + b1_ref[...]
    h1 = jnp.where(pre > 0.0, pre, jnp.exp(jnp.minimum(pre, 0.0)) - 1.0)
    hwa = jnp.dot(h1, wa_ref[...], preferred_element_type=jnp.float32)
    hwb = jnp.dot(h1, wb_ref[...], preferred_element_type=jnp.float32)
    ala, ara, sya, sda = _self_attn(hwa, bdla_ref, bdra_ref, ob_ref,
                                    dup_ref, de4_ref)
    alb, arb, syb, sdb = _self_attn(hwb, bdlb_ref, bdrb_ref, ob_ref,
                                    dup_ref, de4_ref)
    hext2_ref[:, 0:ROWW] = _pack_rows(hwa, ala, ara)
    hext2_ref[:, ROWW:2 * ROWW] = _pack_rows(hwb, alb, arb)
    selfa_ref[...] = jnp.concatenate([sya, sda], axis=1)
    selfb_ref[...] = jnp.concatenate([syb, sdb], axis=1)


def _fin_body(pa_ref, pb_ref, sa_ref, sb_ref, b2_ref, de4_ref, mean_ref,
              out_ref):
    ta = pa_ref[:, 0:80] + sa_ref[...]
    tb = pb_ref[:, 0:80] + sb_ref[...]
    dena = jnp.dot(ta[:, 64:68], de4_ref[...], preferred_element_type=jnp.float32)
    denb = jnp.dot(tb[:, 64:68], de4_ref[...], preferred_element_type=jnp.float32)
    u = ta[:, 0:64] / dena + tb[:, 0:64] / denb
    o = jnp.dot(u, mean_ref[...], preferred_element_type=jnp.float32) + b2_ref[...]
    m = jnp.max(o, axis=-1, keepdims=True)
    z = o - m
    lse = jnp.log(jnp.sum(jnp.exp(z), axis=-1, keepdims=True))
    out_ref[...] = z - lse


def _row_spec(w):
    return pl.BlockSpec((_BLK, w), lambda i: (i, 0))


def _full_spec(r, c):
    return pl.BlockSpec((r, c), lambda i: (0, 0))


def _edge_body(table, srce, dste, out0, out1, accum,
               srcv0, dstv0, srcv1, dstv1, drv0, drv1,
               gbufs, xj0, xi0, xj1, xi1, semI0, semI1, semG0, semG1,
               *, nfold, interleaved):
    f32 = jnp.float32
    cid = lax.axis_index("c")
    sid = lax.axis_index("s")
    lanes = lax.broadcasted_iota(jnp.int32, (16,), 0)
    rot8 = (lanes + 8) % 16
    rot4 = (lanes + 4) % 16

    if interleaved:
        ew = EW2
        base0 = sid * ew
    else:
        ew = EW1
        base0 = (sid * 2 + cid) * ew
    nch = ew // K
    npairs = nch // 2

    def zrow(i, carry):
        for j in range(ROWW // 16):
            xj0[i, pl.ds(j * 16, 16)] = jnp.zeros((16,), f32)
        return carry
    lax.fori_loop(0, K, zrow, 0)
    rb = sid * RPS
    for j in range(RPS // K):
        pltpu.sync_copy(xj0, accum.at[pl.ds(rb + j * K, K)])

    def idx_fetch(c, sv, dv, sem):
        base = base0 + c * K
        pltpu.async_copy(srce.at[pl.ds(base, K)], sv, sem)
        pltpu.async_copy(dste.at[pl.ds(base, K)], dv, sem)

    def idx_wait(sv, dv, sem):
        pltpu.make_async_copy(srce.at[pl.ds(0, K)], sv, sem).wait()
        pltpu.make_async_copy(dste.at[pl.ds(0, K)], dv, sem).wait()

    def gath(sv, dv, xj_, xi_, sem):
        pltpu.async_copy(table.at[sv], xj_, sem)
        pltpu.async_copy(table.at[dv], xi_, sem)

    def gath_wait(xj_, xi_, sem):
        pltpu.make_async_copy(table.at[pl.ds(0, K)], xj_, sem).wait()
        pltpu.make_async_copy(table.at[pl.ds(0, K)], xi_, sem).wait()

    def sel(sv, dv, slot):
        if not interleaved:
            return sv, dv
        svg, dvg = gbufs[slot]
        for j in range(K // 16):
            s16 = sv[pl.ds(j * 16, 16)]
            d16 = dv[pl.ds(j * 16, 16)]
            svg[pl.ds(j * 16, 16)] = s16 + s16 + cid
            dvg[pl.ds(j * 16, 16)] = d16 + d16 + cid
        return svg, dvg

    def mk_drv(sv, dv, drv_):
        for j in range(K // 16):
            s16 = sv[pl.ds(j * 16, 16)]
            d16 = dv[pl.ds(j * 16, 16)]
            drv_[pl.ds(j * 16, 16)] = jnp.where(s16 == d16, jnp.int32(DUMP),
                                                d16)

    def edge_one(xj, xi, e):
        vj = [xj[e, pl.ds(16 * k, 16)] for k in range(4)]
        vi = [xi[e, pl.ds(16 * k, 16)] for k in range(4)]
        p = vj[0] * vi[0]
        for k in range(1, 4):
            p = p + vj[k] * vi[k]
        p = p + p[rot8]
        if nfold == 2:
            p = p + p[rot4]
        s = xj[e, pl.ds(64, 16)] + xi[e, pl.ds(80, 16)]
        alpha = s * (1.0 / (1.0 + jnp.exp(-p)))
        alpha = jnp.maximum(alpha, 0.2 * alpha)
        a = jnp.exp(alpha)
        for k in range(4):
            xj[e, pl.ds(16 * k, 16)] = vj[k] * a
        xj[e, pl.ds(64, 16)] = a

    def edges(xj, xi):
        def e4(i, c):
            for u in range(4):
                edge_one(xj, xi, 4 * i + u)
            return c
        lax.fori_loop(0, K // 4, e4, 0)

    pltpu.sync_copy(srce.at[pl.ds(base0, K)], srcv0)
    pltpu.sync_copy(dste.at[pl.ds(base0, K)], dstv0)
    g0s, g0d = sel(srcv0, dstv0, 0)
    gath(g0s, g0d, xj0, xi0, semG0)
    idx_fetch(1, srcv1, dstv1, semI1)
    plsc.subcore_barrier()

    def pair(p, carry):
        c0 = 2 * p
        more = p < npairs - 1

        idx_wait(srcv1, dstv1, semI1)
        g1s, g1d = sel(srcv1, dstv1, 1)
        gath(g1s, g1d, xj1, xi1, semG1)
        mk_drv(srcv0, dstv0, drv0)
        gath_wait(xj0, xi0, semG0)

        @pl.when(more)
        def _():
            idx_fetch(c0 + 2, srcv0, dstv0, semI0)
        edges(xj0, xi0)
        pltpu.sync_copy(xj0, accum.at[drv0], add=True)

        @pl.when(more)
        def _():
            idx_wait(srcv0, dstv0, semI0)
            gs, gd = sel(srcv0, dstv0, 0)
            gath(gs, gd, xj0, xi0, semG0)
        mk_drv(srcv1, dstv1, drv1)
        gath_wait(xj1, xi1, semG1)

        @pl.when(more)
        def _():
            idx_fetch(c0 + 3, srcv1, dstv1, semI1)
        edges(xj1, xi1)
        pltpu.sync_copy(xj1, accum.at[drv1], add=True)
        return carry
    lax.fori_loop(0, npairs, pair, 0)
    plsc.subcore_barrier()

    @pl.when(cid == 0)
    def _():
        pltpu.sync_copy(accum.at[pl.ds(rb, RPS)], out0.at[pl.ds(rb, RPS)])

    @pl.when(cid == 1)
    def _():
        pltpu.sync_copy(accum.at[pl.ds(rb, RPS)], out1.at[pl.ds(rb, RPS)])


def _make_edge_kernel(interleaved, nfold):
    mesh = plsc.VectorSubcoreMesh(core_axis_name="c", subcore_axis_name="s",
                                  num_cores=2, num_subcores=16)
    f32 = jnp.float32

    def body(table, srce, dste, out0, out1, accum,
             srcv0, dstv0, srcv1, dstv1, drv0, drv1, *rest):
        if interleaved:
            sg0, dg0, sg1, dg1 = rest[:4]
            gbufs = ((sg0, dg0), (sg1, dg1))
            rest = rest[4:]
        else:
            gbufs = None
        (xj0, xi0, xj1, xi1, semI0, semI1, semG0, semG1) = rest
        _edge_body(table, srce, dste, out0, out1, accum,
                   srcv0, dstv0, srcv1, dstv1, drv0, drv1,
                   gbufs, xj0, xi0, xj1, xi1, semI0, semI1, semG0, semG1,
                   nfold=nfold, interleaved=interleaved)

    nivec = 10 if interleaved else 6
    scratch = ([pltpu.VMEM_SHARED((NPAD, ROWW), f32)]
               + [pltpu.VMEM((K,), jnp.int32) for _ in range(nivec)]
               + [pltpu.VMEM((K, ROWW), f32) for _ in range(4)]
               + [pltpu.SemaphoreType.DMA for _ in range(4)])

    return functools.partial(
        pl.kernel,
        out_type=[jax.ShapeDtypeStruct((NPAD, ROWW), f32),
                  jax.ShapeDtypeStruct((NPAD, ROWW), f32)],
        mesh=mesh,
        scratch_types=scratch,
    )(body)


_edge_kernel_cache = {}


def _edge_kernel_for(interleaved, nfold):
    kk = (interleaved, nfold)
    if kk not in _edge_kernel_cache:
        _edge_kernel_cache[kk] = _make_edge_kernel(interleaved, nfold)
    return _edge_kernel_cache[kk]


def _cm_blockdiag(att2, nh):
    ch = att2.shape[1]
    return (att2.T.reshape(ch, nh, 1)
            * jnp.eye(nh, dtype=jnp.float32).reshape(1, nh, nh)).reshape(ch * nh, nh)


def kernel(x, edge_index, W1, att_l1, att_r1, b1, W2, att_l2, att_r2, b2):
    f32 = jnp.float32
    i32 = jnp.int32
    src = edge_index[0]
    dst = edge_index[1]

    zpad = jnp.zeros((EPAD - E,), i32)
    srcp = jnp.concatenate([src, zpad])
    dstp = jnp.concatenate([dst, zpad])

    cm1 = (jnp.arange(HC1, dtype=i32) % H) * C1 + jnp.arange(HC1, dtype=i32) // H
    w1cm = W1[:, cm1]
    at_l1 = att_l1.reshape(H, C1)
    at_r1 = att_r1.reshape(H, C1)
    bdl1 = _cm_blockdiag(at_l1, H)
    bdr1 = _cm_blockdiag(at_r1, H)
    ob1 = jnp.kron(jnp.ones((C1, 1), f32), jnp.eye(H, dtype=f32))
    dup8 = jnp.kron(jnp.ones((1, 2), f32), jnp.eye(H, dtype=f32))
    de8 = jnp.kron(jnp.ones((1, C1), f32), jnp.eye(H, dtype=f32))

    cmh = (jnp.arange(64, dtype=i32) % 4) * NCLS + jnp.arange(64, dtype=i32) // 4
    w2r = W2[cm1, :]
    w2a = w2r[:, cmh]
    w2b = w2r[:, cmh + 4 * NCLS]
    at_l2 = att_l2.reshape(H, NCLS)
    at_r2 = att_r2.reshape(H, NCLS)
    bdl2a = _cm_blockdiag(at_l2[0:4], 4)
    bdr2a = _cm_blockdiag(at_r2[0:4], 4)
    bdl2b = _cm_blockdiag(at_l2[4:8], 4)
    bdr2b = _cm_blockdiag(at_r2[4:8], 4)
    ob2 = jnp.kron(jnp.ones((NCLS, 1), f32), jnp.eye(4, dtype=f32))
    dup4 = jnp.kron(jnp.ones((1, 4), f32), jnp.eye(4, dtype=f32))
    de4 = jnp.kron(jnp.ones((1, NCLS), f32), jnp.eye(4, dtype=f32))
    meancm = jnp.kron(jnp.eye(NCLS, dtype=f32), jnp.full((4, 1), 1.0 / H, f32))

    hext1, self1 = pl.pallas_call(
        _prep1_body,
        grid=(_GRID,),
        in_specs=[_row_spec(D), _full_spec(D, HC1), _full_spec(HC1, H),
                  _full_spec(HC1, H), _full_spec(HC1, H), _full_spec(H, 16),
                  _full_spec(H, HC1)],
        out_specs=[_row_spec(ROWW), _row_spec(80)],
        out_shape=[jax.ShapeDtypeStruct((N, ROWW), f32),
                   jax.ShapeDtypeStruct((N, 80), f32)],
    )(x, w1cm, bdl1, bdr1, ob1, dup8, de8)

    p10, p11 = _edge_kernel_for(False, 1)(hext1, srcp, dstp)

    hext2, selfa, selfb = pl.pallas_call(
        _mid_body,
        grid=(_GRID,),
        in_specs=[_row_spec(ROWW), _row_spec(ROWW), _row_spec(80),
                  _full_spec(1, HC1), _full_spec(HC1, 64), _full_spec(HC1, 64),
                  _full_spec(64, 4), _full_spec(64, 4), _full_spec(64, 4),
                  _full_spec(64, 4), _full_spec(64, 4), _full_spec(4, 16),
                  _full_spec(4, 64), _full_spec(H, HC1)],
        out_specs=[_row_spec(2 * ROWW), _row_spec(80), _row_spec(80)],
        out_shape=[jax.ShapeDtypeStruct((N, 2 * ROWW), f32),
                   jax.ShapeDtypeStruct((N, 80), f32),
                   jax.ShapeDtypeStruct((N, 80), f32)],
    )(p10, p11, self1, b1[cm1].reshape(1, HC1), w2a, w2b,
      bdl2a, bdr2a, bdl2b, bdr2b, ob2, dup4, de4, de8)

    p20, p21 = _edge_kernel_for(True, 2)(hext2.reshape(2 * N, ROWW),
                                         srcp, dstp)

    out = pl.pallas_call(
        _fin_body,
        grid=(_GRID,),
        in_specs=[_row_spec(ROWW), _row_spec(ROWW), _row_spec(80),
                  _row_spec(80), _full_spec(1, NCLS), _full_spec(4, 64),
                  _full_spec(64, NCLS)],
        out_specs=[_row_spec(NCLS)],
        out_shape=[jax.ShapeDtypeStruct((N, NCLS), f32)],
    )(p20, p21, selfa, selfb, b2.reshape(1, NCLS), de4, meancm)[0]

    return out, jnp.float32(0.0)


# speedup vs baseline: 1.0462x; 1.0020x over previous
import functools

import jax
import jax.numpy as jnp
from jax import lax
from jax.experimental import pallas as pl
from jax.experimental.pallas import tpu as pltpu
from jax.experimental.pallas import tpu_sc as plsc

N = 10000
E = 320000
D = 128
H = 8
C1 = 8
NCLS = 16
HC1 = H * C1
ROWW = 128

K = 64
EPAD = 327680
EW1 = EPAD // 32
EW2 = EPAD // 16
NPAD = 10240
RPS = NPAD // 16
DUMP = N

_BLK = 1000
_GRID = N // _BLK


def _self_attn(hw, bdl_ref, bdr_ref, ob_ref, dup_ref, de_ref):
    al = jnp.dot(hw, bdl_ref[...], preferred_element_type=jnp.float32)
    ar = jnp.dot(hw, bdr_ref[...], preferred_element_type=jnp.float32)
    lg = jnp.dot(hw * hw, ob_ref[...], preferred_element_type=jnp.float32)
    alpha = (al + ar) * (1.0 / (1.0 + jnp.exp(-lg)))
    alpha = jnp.where(alpha >= 0.0, alpha, 0.2 * alpha)
    a_s = jnp.exp(alpha)
    aldup = jnp.dot(al, dup_ref[...], preferred_element_type=jnp.float32)
    ardup = jnp.dot(ar, dup_ref[...], preferred_element_type=jnp.float32)
    selfy = hw * jnp.dot(a_s, de_ref[...], preferred_element_type=jnp.float32)
    selfd = jnp.dot(a_s, dup_ref[...], preferred_element_type=jnp.float32)
    return aldup, ardup, selfy, selfd


def _pack_rows(hw, aldup, ardup):
    z = jnp.zeros((hw.shape[0], 32), jnp.float32)
    return jnp.concatenate([hw, aldup, ardup, z], axis=1)


def _pack(hext_ref, self_ref, hw, aldup, ardup, selfy, selfd):
    hext_ref[...] = _pack_rows(hw, aldup, ardup)
    self_ref[...] = jnp.concatenate([selfy, selfd], axis=1)


def _prep1_body(x_ref, w_ref, bdl_ref, bdr_ref, ob_ref, dup_ref, de_ref,
                hext_ref, self_ref):
    hw = jnp.dot(x_ref[...], w_ref[...], preferred_element_type=jnp.float32)
    aldup, ardup, selfy, selfd = _self_attn(hw, bdl_ref, bdr_ref, ob_ref,
                                            dup_ref, de_ref)
    _pack(hext_ref, self_ref, hw, aldup, ardup, selfy, selfd)


def _mid_body(p0_ref, p1_ref, s1_ref, b1_ref, wa_ref, wb_ref,
              bdla_ref, bdra_ref, bdlb_ref, bdrb_ref, ob_ref,
              dup_ref, de4_ref, de8_ref, hext2_ref,
              selfa_ref, selfb_ref):
    t = p0_ref[:, 0:80] + p1_ref[:, 0:80] + s1_ref[...]
    den = jnp.dot(t[:, 64:72], de8_ref[...], preferred_element_type=jnp.float32)
    pre = t[:, 0:64] / den + b1_ref[...]
    h1 = jnp.where(pre > 0.0, pre, jnp.exp(jnp.minimum(pre, 0.0)) - 1.0)
    hwa = jnp.dot(h1, wa_ref[...], preferred_element_type=jnp.float32)
    hwb = jnp.dot(h1, wb_ref[...], preferred_element_type=jnp.float32)
    ala, ara, sya, sda = _self_attn(hwa, bdla_ref, bdra_ref, ob_ref,
                                    dup_ref, de4_ref)
    alb, arb, syb, sdb = _self_attn(hwb, bdlb_ref, bdrb_ref, ob_ref,
                                    dup_ref, de4_ref)
    hext2_ref[:, 0:ROWW] = _pack_rows(hwa, ala, ara)
    hext2_ref[:, ROWW:2 * ROWW] = _pack_rows(hwb, alb, arb)
    selfa_ref[...] = jnp.concatenate([sya, sda], axis=1)
    selfb_ref[...] = jnp.concatenate([syb, sdb], axis=1)


def _fin_body(pa_ref, pb_ref, sa_ref, sb_ref, b2_ref, de4_ref, mean_ref,
              out_ref):
    ta = pa_ref[:, 0:80] + sa_ref[...]
    tb = pb_ref[:, 0:80] + sb_ref[...]
    dena = jnp.dot(ta[:, 64:68], de4_ref[...], preferred_element_type=jnp.float32)
    denb = jnp.dot(tb[:, 64:68], de4_ref[...], preferred_element_type=jnp.float32)
    u = ta[:, 0:64] / dena + tb[:, 0:64] / denb
    o = jnp.dot(u, mean_ref[...], preferred_element_type=jnp.float32) + b2_ref[...]
    m = jnp.max(o, axis=-1, keepdims=True)
    z = o - m
    lse = jnp.log(jnp.sum(jnp.exp(z), axis=-1, keepdims=True))
    out_ref[...] = z - lse


def _row_spec(w):
    return pl.BlockSpec((_BLK, w), lambda i: (i, 0))


def _full_spec(r, c):
    return pl.BlockSpec((r, c), lambda i: (0, 0))


def _edge_body(table, srce, dste, out0, out1, accum,
               srcv0, dstv0, srcv1, dstv1, drv0, drv1,
               gbufs, xj0, xi0, xj1, xi1, semI0, semI1, semG0, semG1,
               *, nfold, interleaved):
    f32 = jnp.float32
    cid = lax.axis_index("c")
    sid = lax.axis_index("s")
    lanes = lax.broadcasted_iota(jnp.int32, (16,), 0)
    rot8 = (lanes + 8) % 16
    rot4 = (lanes + 4) % 16

    if interleaved:
        ew = EW2
        base0 = sid * ew
    else:
        ew = EW1
        base0 = (sid * 2 + cid) * ew
    nch = ew // K
    npairs = nch // 2

    def zrow(i, carry):
        for j in range(ROWW // 16):
            xj0[i, pl.ds(j * 16, 16)] = jnp.zeros((16,), f32)
        return carry
    lax.fori_loop(0, K, zrow, 0)
    rb = sid * RPS
    for j in range(RPS // K):
        pltpu.sync_copy(xj0, accum.at[pl.ds(rb + j * K, K)])

    def idx_fetch(c, sv, dv, sem):
        base = base0 + c * K
        pltpu.async_copy(srce.at[pl.ds(base, K)], sv, sem)
        pltpu.async_copy(dste.at[pl.ds(base, K)], dv, sem)

    def idx_wait(sv, dv, sem):
        pltpu.make_async_copy(srce.at[pl.ds(0, K)], sv, sem).wait()
        pltpu.make_async_copy(dste.at[pl.ds(0, K)], dv, sem).wait()

    def gath(sv, dv, xj_, xi_, sem):
        pltpu.async_copy(table.at[sv], xj_, sem)
        pltpu.async_copy(table.at[dv], xi_, sem)

    def gath_wait(xj_, xi_, sem):
        pltpu.make_async_copy(table.at[pl.ds(0, K)], xj_, sem).wait()
        pltpu.make_async_copy(table.at[pl.ds(0, K)], xi_, sem).wait()

    def sel(sv, dv, slot):
        if not interleaved:
            return sv, dv
        svg, dvg = gbufs[slot]
        for j in range(K // 16):
            s16 = sv[pl.ds(j * 16, 16)]
            d16 = dv[pl.ds(j * 16, 16)]
            svg[pl.ds(j * 16, 16)] = s16 + s16 + cid
            dvg[pl.ds(j * 16, 16)] = d16 + d16 + cid
        return svg, dvg

    def mk_drv(sv, dv, drv_):
        for j in range(K // 16):
            s16 = sv[pl.ds(j * 16, 16)]
            d16 = dv[pl.ds(j * 16, 16)]
            drv_[pl.ds(j * 16, 16)] = jnp.where(s16 == d16, jnp.int32(DUMP),
                                                d16)

    def edge_one(xj, xi, e):
        vj = [xj[e, pl.ds(16 * k, 16)] for k in range(4)]
        vi = [xi[e, pl.ds(16 * k, 16)] for k in range(4)]
        p = vj[0] * vi[0]
        for k in range(1, 4):
            p = p + vj[k] * vi[k]
        p = p + p[rot8]
        if nfold == 2:
            p = p + p[rot4]
        s = xj[e, pl.ds(64, 16)] + xi[e, pl.ds(80, 16)]
        alpha = s * (1.0 / (1.0 + jnp.exp(-p)))
        alpha = jnp.maximum(alpha, 0.2 * alpha)
        a = jnp.exp(alpha)
        for k in range(4):
            xj[e, pl.ds(16 * k, 16)] = vj[k] * a
        xj[e, pl.ds(64, 16)] = a

    def edges(xj, xi):
        def e8(i, c):
            for u in range(8):
                edge_one(xj, xi, 8 * i + u)
            return c
        lax.fori_loop(0, K // 8, e8, 0)

    pltpu.sync_copy(srce.at[pl.ds(base0, K)], srcv0)
    pltpu.sync_copy(dste.at[pl.ds(base0, K)], dstv0)
    g0s, g0d = sel(srcv0, dstv0, 0)
    gath(g0s, g0d, xj0, xi0, semG0)
    idx_fetch(1, srcv1, dstv1, semI1)
    plsc.subcore_barrier()

    def pair(p, carry):
        c0 = 2 * p
        more = p < npairs - 1

        idx_wait(srcv1, dstv1, semI1)
        g1s, g1d = sel(srcv1, dstv1, 1)
        gath(g1s, g1d, xj1, xi1, semG1)
        mk_drv(srcv0, dstv0, drv0)
        gath_wait(xj0, xi0, semG0)

        @pl.when(more)
        def _():
            idx_fetch(c0 + 2, srcv0, dstv0, semI0)
        edges(xj0, xi0)
        pltpu.sync_copy(xj0, accum.at[drv0], add=True)

        @pl.when(more)
        def _():
            idx_wait(srcv0, dstv0, semI0)
            gs, gd = sel(srcv0, dstv0, 0)
            gath(gs, gd, xj0, xi0, semG0)
        mk_drv(srcv1, dstv1, drv1)
        gath_wait(xj1, xi1, semG1)

        @pl.when(more)
        def _():
            idx_fetch(c0 + 3, srcv1, dstv1, semI1)
        edges(xj1, xi1)
        pltpu.sync_copy(xj1, accum.at[drv1], add=True)
        return carry
    lax.fori_loop(0, npairs, pair, 0)
    plsc.subcore_barrier()

    @pl.when(cid == 0)
    def _():
        pltpu.sync_copy(accum.at[pl.ds(rb, RPS)], out0.at[pl.ds(rb, RPS)])

    @pl.when(cid == 1)
    def _():
        pltpu.sync_copy(accum.at[pl.ds(rb, RPS)], out1.at[pl.ds(rb, RPS)])


def _make_edge_kernel(interleaved, nfold):
    mesh = plsc.VectorSubcoreMesh(core_axis_name="c", subcore_axis_name="s",
                                  num_cores=2, num_subcores=16)
    f32 = jnp.float32

    def body(table, srce, dste, out0, out1, accum,
             srcv0, dstv0, srcv1, dstv1, drv0, drv1, *rest):
        if interleaved:
            sg0, dg0, sg1, dg1 = rest[:4]
            gbufs = ((sg0, dg0), (sg1, dg1))
            rest = rest[4:]
        else:
            gbufs = None
        (xj0, xi0, xj1, xi1, semI0, semI1, semG0, semG1) = rest
        _edge_body(table, srce, dste, out0, out1, accum,
                   srcv0, dstv0, srcv1, dstv1, drv0, drv1,
                   gbufs, xj0, xi0, xj1, xi1, semI0, semI1, semG0, semG1,
                   nfold=nfold, interleaved=interleaved)

    nivec = 10 if interleaved else 6
    scratch = ([pltpu.VMEM_SHARED((NPAD, ROWW), f32)]
               + [pltpu.VMEM((K,), jnp.int32) for _ in range(nivec)]
               + [pltpu.VMEM((K, ROWW), f32) for _ in range(4)]
               + [pltpu.SemaphoreType.DMA for _ in range(4)])

    return functools.partial(
        pl.kernel,
        out_type=[jax.ShapeDtypeStruct((NPAD, ROWW), f32),
                  jax.ShapeDtypeStruct((NPAD, ROWW), f32)],
        mesh=mesh,
        scratch_types=scratch,
    )(body)


_edge_kernel_cache = {}


def _edge_kernel_for(interleaved, nfold):
    kk = (interleaved, nfold)
    if kk not in _edge_kernel_cache:
        _edge_kernel_cache[kk] = _make_edge_kernel(interleaved, nfold)
    return _edge_kernel_cache[kk]


def _cm_blockdiag(att2, nh):
    ch = att2.shape[1]
    return (att2.T.reshape(ch, nh, 1)
            * jnp.eye(nh, dtype=jnp.float32).reshape(1, nh, nh)).reshape(ch * nh, nh)


def kernel(x, edge_index, W1, att_l1, att_r1, b1, W2, att_l2, att_r2, b2):
    f32 = jnp.float32
    i32 = jnp.int32
    src = edge_index[0]
    dst = edge_index[1]

    zpad = jnp.zeros((EPAD - E,), i32)
    srcp = jnp.concatenate([src, zpad])
    dstp = jnp.concatenate([dst, zpad])

    cm1 = (jnp.arange(HC1, dtype=i32) % H) * C1 + jnp.arange(HC1, dtype=i32) // H
    w1cm = W1[:, cm1]
    at_l1 = att_l1.reshape(H, C1)
    at_r1 = att_r1.reshape(H, C1)
    bdl1 = _cm_blockdiag(at_l1, H)
    bdr1 = _cm_blockdiag(at_r1, H)
    ob1 = jnp.kron(jnp.ones((C1, 1), f32), jnp.eye(H, dtype=f32))
    dup8 = jnp.kron(jnp.ones((1, 2), f32), jnp.eye(H, dtype=f32))
    de8 = jnp.kron(jnp.ones((1, C1), f32), jnp.eye(H, dtype=f32))

    cmh = (jnp.arange(64, dtype=i32) % 4) * NCLS + jnp.arange(64, dtype=i32) // 4
    w2r = W2[cm1, :]
    w2a = w2r[:, cmh]
    w2b = w2r[:, cmh + 4 * NCLS]
    at_l2 = att_l2.reshape(H, NCLS)
    at_r2 = att_r2.reshape(H, NCLS)
    bdl2a = _cm_blockdiag(at_l2[0:4], 4)
    bdr2a = _cm_blockdiag(at_r2[0:4], 4)
    bdl2b = _cm_blockdiag(at_l2[4:8], 4)
    bdr2b = _cm_blockdiag(at_r2[4:8], 4)
    ob2 = jnp.kron(jnp.ones((NCLS, 1), f32), jnp.eye(4, dtype=f32))
    dup4 = jnp.kron(jnp.ones((1, 4), f32), jnp.eye(4, dtype=f32))
    de4 = jnp.kron(jnp.ones((1, NCLS), f32), jnp.eye(4, dtype=f32))
    meancm = jnp.kron(jnp.eye(NCLS, dtype=f32), jnp.full((4, 1), 1.0 / H, f32))

    hext1, self1 = pl.pallas_call(
        _prep1_body,
        grid=(_GRID,),
        in_specs=[_row_spec(D), _full_spec(D, HC1), _full_spec(HC1, H),
                  _full_spec(HC1, H), _full_spec(HC1, H), _full_spec(H, 16),
                  _full_spec(H, HC1)],
        out_specs=[_row_spec(ROWW), _row_spec(80)],
        out_shape=[jax.ShapeDtypeStruct((N, ROWW), f32),
                   jax.ShapeDtypeStruct((N, 80), f32)],
    )(x, w1cm, bdl1, bdr1, ob1, dup8, de8)

    p10, p11 = _edge_kernel_for(False, 1)(hext1, srcp, dstp)

    hext2, selfa, selfb = pl.pallas_call(
        _mid_body,
        grid=(_GRID,),
        in_specs=[_row_spec(ROWW), _row_spec(ROWW), _row_spec(80),
                  _full_spec(1, HC1), _full_spec(HC1, 64), _full_spec(HC1, 64),
                  _full_spec(64, 4), _full_spec(64, 4), _full_spec(64, 4),
                  _full_spec(64, 4), _full_spec(64, 4), _full_spec(4, 16),
                  _full_spec(4, 64), _full_spec(H, HC1)],
        out_specs=[_row_spec(2 * ROWW), _row_spec(80), _row_spec(80)],
        out_shape=[jax.ShapeDtypeStruct((N, 2 * ROWW), f32),
                   jax.ShapeDtypeStruct((N, 80), f32),
                   jax.ShapeDtypeStruct((N, 80), f32)],
    )(p10, p11, self1, b1[cm1].reshape(1, HC1), w2a, w2b,
      bdl2a, bdr2a, bdl2b, bdr2b, ob2, dup4, de4, de8)

    p20, p21 = _edge_kernel_for(True, 2)(hext2.reshape(2 * N, ROWW),
                                         srcp, dstp)

    out = pl.pallas_call(
        _fin_body,
        grid=(_GRID,),
        in_specs=[_row_spec(ROWW), _row_spec(ROWW), _row_spec(80),
                  _row_spec(80), _full_spec(1, NCLS), _full_spec(4, 64),
                  _full_spec(64, NCLS)],
        out_specs=[_row_spec(NCLS)],
        out_shape=[jax.ShapeDtypeStruct((N, NCLS), f32)],
    )(p20, p21, selfa, selfb, b2.reshape(1, NCLS), de4, meancm)[0]

    return out, jnp.float32(0.0)


# speedup vs baseline: 1.0487x; 1.0024x over previous
import functools

import jax
import jax.numpy as jnp
from jax import lax
from jax.experimental import pallas as pl
from jax.experimental.pallas import tpu as pltpu
from jax.experimental.pallas import tpu_sc as plsc

N = 10000
E = 320000
D = 128
H = 8
C1 = 8
NCLS = 16
HC1 = H * C1
ROWW = 128

K = 80
EPAD = 327680
EW1 = EPAD // 32
EW2 = EPAD // 16
NPAD = 10240
RPS = NPAD // 16
DUMP = N

_BLK = 1000
_GRID = N // _BLK


def _self_attn(hw, bdl_ref, bdr_ref, ob_ref, dup_ref, de_ref):
    al = jnp.dot(hw, bdl_ref[...], preferred_element_type=jnp.float32)
    ar = jnp.dot(hw, bdr_ref[...], preferred_element_type=jnp.float32)
    lg = jnp.dot(hw * hw, ob_ref[...], preferred_element_type=jnp.float32)
    alpha = (al + ar) * (1.0 / (1.0 + jnp.exp(-lg)))
    alpha = jnp.where(alpha >= 0.0, alpha, 0.2 * alpha)
    a_s = jnp.exp(alpha)
    aldup = jnp.dot(al, dup_ref[...], preferred_element_type=jnp.float32)
    ardup = jnp.dot(ar, dup_ref[...], preferred_element_type=jnp.float32)
    selfy = hw * jnp.dot(a_s, de_ref[...], preferred_element_type=jnp.float32)
    selfd = jnp.dot(a_s, dup_ref[...], preferred_element_type=jnp.float32)
    return aldup, ardup, selfy, selfd


def _pack_rows(hw, aldup, ardup):
    z = jnp.zeros((hw.shape[0], 32), jnp.float32)
    return jnp.concatenate([hw, aldup, ardup, z], axis=1)


def _pack(hext_ref, self_ref, hw, aldup, ardup, selfy, selfd):
    hext_ref[...] = _pack_rows(hw, aldup, ardup)
    self_ref[...] = jnp.concatenate([selfy, selfd], axis=1)


def _prep1_body(x_ref, w_ref, bdl_ref, bdr_ref, ob_ref, dup_ref, de_ref,
                hext_ref, self_ref):
    hw = jnp.dot(x_ref[...], w_ref[...], preferred_element_type=jnp.float32)
    aldup, ardup, selfy, selfd = _self_attn(hw, bdl_ref, bdr_ref, ob_ref,
                                            dup_ref, de_ref)
    _pack(hext_ref, self_ref, hw, aldup, ardup, selfy, selfd)


def _mid_body(p0_ref, p1_ref, s1_ref, b1_ref, wa_ref, wb_ref,
              bdla_ref, bdra_ref, bdlb_ref, bdrb_ref, ob_ref,
              dup_ref, de4_ref, de8_ref, hext2_ref,
              selfa_ref, selfb_ref):
    t = p0_ref[:, 0:80] + p1_ref[:, 0:80] + s1_ref[...]
    den = jnp.dot(t[:, 64:72], de8_ref[...], preferred_element_type=jnp.float32)
    pre = t[:, 0:64] / den + b1_ref[...]
    h1 = jnp.where(pre > 0.0, pre, jnp.exp(jnp.minimum(pre, 0.0)) - 1.0)
    hwa = jnp.dot(h1, wa_ref[...], preferred_element_type=jnp.float32)
    hwb = jnp.dot(h1, wb_ref[...], preferred_element_type=jnp.float32)
    ala, ara, sya, sda = _self_attn(hwa, bdla_ref, bdra_ref, ob_ref,
                                    dup_ref, de4_ref)
    alb, arb, syb, sdb = _self_attn(hwb, bdlb_ref, bdrb_ref, ob_ref,
                                    dup_ref, de4_ref)
    hext2_ref[:, 0:ROWW] = _pack_rows(hwa, ala, ara)
    hext2_ref[:, ROWW:2 * ROWW] = _pack_rows(hwb, alb, arb)
    selfa_ref[...] = jnp.concatenate([sya, sda], axis=1)
    selfb_ref[...] = jnp.concatenate([syb, sdb], axis=1)


def _fin_body(pa_ref, pb_ref, sa_ref, sb_ref, b2_ref, de4_ref, mean_ref,
              out_ref):
    ta = pa_ref[:, 0:80] + sa_ref[...]
    tb = pb_ref[:, 0:80] + sb_ref[...]
    dena = jnp.dot(ta[:, 64:68], de4_ref[...], preferred_element_type=jnp.float32)
    denb = jnp.dot(tb[:, 64:68], de4_ref[...], preferred_element_type=jnp.float32)
    u = ta[:, 0:64] / dena + tb[:, 0:64] / denb
    o = jnp.dot(u, mean_ref[...], preferred_element_type=jnp.float32) + b2_ref[...]
    m = jnp.max(o, axis=-1, keepdims=True)
    z = o - m
    lse = jnp.log(jnp.sum(jnp.exp(z), axis=-1, keepdims=True))
    out_ref[...] = z - lse


def _row_spec(w):
    return pl.BlockSpec((_BLK, w), lambda i: (i, 0))


def _full_spec(r, c):
    return pl.BlockSpec((r, c), lambda i: (0, 0))


def _edge_body(table, srce, dste, out0, out1, accum,
               srcv0, dstv0, srcv1, dstv1, drv0, drv1,
               gbufs, xj0, xi0, xj1, xi1, semI0, semI1, semG0, semG1,
               *, nfold, interleaved):
    f32 = jnp.float32
    cid = lax.axis_index("c")
    sid = lax.axis_index("s")
    lanes = lax.broadcasted_iota(jnp.int32, (16,), 0)
    rot8 = (lanes + 8) % 16
    rot4 = (lanes + 4) % 16

    if interleaved:
        ew = EW2
        base0 = sid * ew
    else:
        ew = EW1
        base0 = (sid * 2 + cid) * ew
    nch = ew // K
    npairs = nch // 2

    def zrow(i, carry):
        for j in range(ROWW // 16):
            xj0[i, pl.ds(j * 16, 16)] = jnp.zeros((16,), f32)
        return carry
    lax.fori_loop(0, K, zrow, 0)
    rb = sid * RPS
    for j in range(RPS // K):
        pltpu.sync_copy(xj0, accum.at[pl.ds(rb + j * K, K)])

    def idx_fetch(c, sv, dv, sem):
        base = base0 + c * K
        pltpu.async_copy(srce.at[pl.ds(base, K)], sv, sem)
        pltpu.async_copy(dste.at[pl.ds(base, K)], dv, sem)

    def idx_wait(sv, dv, sem):
        pltpu.make_async_copy(srce.at[pl.ds(0, K)], sv, sem).wait()
        pltpu.make_async_copy(dste.at[pl.ds(0, K)], dv, sem).wait()

    def gath(sv, dv, xj_, xi_, sem):
        pltpu.async_copy(table.at[sv], xj_, sem)
        pltpu.async_copy(table.at[dv], xi_, sem)

    def gath_wait(xj_, xi_, sem):
        pltpu.make_async_copy(table.at[pl.ds(0, K)], xj_, sem).wait()
        pltpu.make_async_copy(table.at[pl.ds(0, K)], xi_, sem).wait()

    def sel(sv, dv, slot):
        if not interleaved:
            return sv, dv
        svg, dvg = gbufs[slot]
        for j in range(K // 16):
            s16 = sv[pl.ds(j * 16, 16)]
            d16 = dv[pl.ds(j * 16, 16)]
            svg[pl.ds(j * 16, 16)] = s16 + s16 + cid
            dvg[pl.ds(j * 16, 16)] = d16 + d16 + cid
        return svg, dvg

    def mk_drv(sv, dv, drv_):
        for j in range(K // 16):
            s16 = sv[pl.ds(j * 16, 16)]
            d16 = dv[pl.ds(j * 16, 16)]
            drv_[pl.ds(j * 16, 16)] = jnp.where(s16 == d16, jnp.int32(DUMP),
                                                d16)

    def edge_one(xj, xi, e):
        vj = [xj[e, pl.ds(16 * k, 16)] for k in range(4)]
        vi = [xi[e, pl.ds(16 * k, 16)] for k in range(4)]
        p = vj[0] * vi[0]
        for k in range(1, 4):
            p = p + vj[k] * vi[k]
        p = p + p[rot8]
        if nfold == 2:
            p = p + p[rot4]
        s = xj[e, pl.ds(64, 16)] + xi[e, pl.ds(80, 16)]
        alpha = s * (1.0 / (1.0 + jnp.exp(-p)))
        alpha = jnp.maximum(alpha, 0.2 * alpha)
        a = jnp.exp(alpha)
        for k in range(4):
            xj[e, pl.ds(16 * k, 16)] = vj[k] * a
        xj[e, pl.ds(64, 16)] = a

    def edges(xj, xi):
        def e8(i, c):
            for u in range(8):
                edge_one(xj, xi, 8 * i + u)
            return c
        lax.fori_loop(0, K // 8, e8, 0)

    pltpu.sync_copy(srce.at[pl.ds(base0, K)], srcv0)
    pltpu.sync_copy(dste.at[pl.ds(base0, K)], dstv0)
    g0s, g0d = sel(srcv0, dstv0, 0)
    gath(g0s, g0d, xj0, xi0, semG0)
    idx_fetch(1, srcv1, dstv1, semI1)
    plsc.subcore_barrier()

    def pair(p, carry):
        c0 = 2 * p
        more = p < npairs - 1

        idx_wait(srcv1, dstv1, semI1)
        g1s, g1d = sel(srcv1, dstv1, 1)
        gath(g1s, g1d, xj1, xi1, semG1)
        mk_drv(srcv0, dstv0, drv0)
        gath_wait(xj0, xi0, semG0)

        @pl.when(more)
        def _():
            idx_fetch(c0 + 2, srcv0, dstv0, semI0)
        edges(xj0, xi0)
        pltpu.sync_copy(xj0, accum.at[drv0], add=True)

        @pl.when(more)
        def _():
            idx_wait(srcv0, dstv0, semI0)
            gs, gd = sel(srcv0, dstv0, 0)
            gath(gs, gd, xj0, xi0, semG0)
        mk_drv(srcv1, dstv1, drv1)
        gath_wait(xj1, xi1, semG1)

        @pl.when(more)
        def _():
            idx_fetch(c0 + 3, srcv1, dstv1, semI1)
        edges(xj1, xi1)
        pltpu.sync_copy(xj1, accum.at[drv1], add=True)
        return carry
    lax.fori_loop(0, npairs, pair, 0)
    plsc.subcore_barrier()

    @pl.when(cid == 0)
    def _():
        pltpu.sync_copy(accum.at[pl.ds(rb, RPS)], out0.at[pl.ds(rb, RPS)])

    @pl.when(cid == 1)
    def _():
        pltpu.sync_copy(accum.at[pl.ds(rb, RPS)], out1.at[pl.ds(rb, RPS)])


def _make_edge_kernel(interleaved, nfold):
    mesh = plsc.VectorSubcoreMesh(core_axis_name="c", subcore_axis_name="s",
                                  num_cores=2, num_subcores=16)
    f32 = jnp.float32

    def body(table, srce, dste, out0, out1, accum,
             srcv0, dstv0, srcv1, dstv1, drv0, drv1, *rest):
        if interleaved:
            sg0, dg0, sg1, dg1 = rest[:4]
            gbufs = ((sg0, dg0), (sg1, dg1))
            rest = rest[4:]
        else:
            gbufs = None
        (xj0, xi0, xj1, xi1, semI0, semI1, semG0, semG1) = rest
        _edge_body(table, srce, dste, out0, out1, accum,
                   srcv0, dstv0, srcv1, dstv1, drv0, drv1,
                   gbufs, xj0, xi0, xj1, xi1, semI0, semI1, semG0, semG1,
                   nfold=nfold, interleaved=interleaved)

    nivec = 10 if interleaved else 6
    scratch = ([pltpu.VMEM_SHARED((NPAD, ROWW), f32)]
               + [pltpu.VMEM((K,), jnp.int32) for _ in range(nivec)]
               + [pltpu.VMEM((K, ROWW), f32) for _ in range(4)]
               + [pltpu.SemaphoreType.DMA for _ in range(4)])

    return functools.partial(
        pl.kernel,
        out_type=[jax.ShapeDtypeStruct((NPAD, ROWW), f32),
                  jax.ShapeDtypeStruct((NPAD, ROWW), f32)],
        mesh=mesh,
        scratch_types=scratch,
    )(body)


_edge_kernel_cache = {}


def _edge_kernel_for(interleaved, nfold):
    kk = (interleaved, nfold)
    if kk not in _edge_kernel_cache:
        _edge_kernel_cache[kk] = _make_edge_kernel(interleaved, nfold)
    return _edge_kernel_cache[kk]


def _cm_blockdiag(att2, nh):
    ch = att2.shape[1]
    return (att2.T.reshape(ch, nh, 1)
            * jnp.eye(nh, dtype=jnp.float32).reshape(1, nh, nh)).reshape(ch * nh, nh)


def kernel(x, edge_index, W1, att_l1, att_r1, b1, W2, att_l2, att_r2, b2):
    f32 = jnp.float32
    i32 = jnp.int32
    src = edge_index[0]
    dst = edge_index[1]

    zpad = jnp.zeros((EPAD - E,), i32)
    srcp = jnp.concatenate([src, zpad])
    dstp = jnp.concatenate([dst, zpad])

    cm1 = (jnp.arange(HC1, dtype=i32) % H) * C1 + jnp.arange(HC1, dtype=i32) // H
    w1cm = W1[:, cm1]
    at_l1 = att_l1.reshape(H, C1)
    at_r1 = att_r1.reshape(H, C1)
    bdl1 = _cm_blockdiag(at_l1, H)
    bdr1 = _cm_blockdiag(at_r1, H)
    ob1 = jnp.kron(jnp.ones((C1, 1), f32), jnp.eye(H, dtype=f32))
    dup8 = jnp.kron(jnp.ones((1, 2), f32), jnp.eye(H, dtype=f32))
    de8 = jnp.kron(jnp.ones((1, C1), f32), jnp.eye(H, dtype=f32))

    cmh = (jnp.arange(64, dtype=i32) % 4) * NCLS + jnp.arange(64, dtype=i32) // 4
    w2r = W2[cm1, :]
    w2a = w2r[:, cmh]
    w2b = w2r[:, cmh + 4 * NCLS]
    at_l2 = att_l2.reshape(H, NCLS)
    at_r2 = att_r2.reshape(H, NCLS)
    bdl2a = _cm_blockdiag(at_l2[0:4], 4)
    bdr2a = _cm_blockdiag(at_r2[0:4], 4)
    bdl2b = _cm_blockdiag(at_l2[4:8], 4)
    bdr2b = _cm_blockdiag(at_r2[4:8], 4)
    ob2 = jnp.kron(jnp.ones((NCLS, 1), f32), jnp.eye(4, dtype=f32))
    dup4 = jnp.kron(jnp.ones((1, 4), f32), jnp.eye(4, dtype=f32))
    de4 = jnp.kron(jnp.ones((1, NCLS), f32), jnp.eye(4, dtype=f32))
    meancm = jnp.kron(jnp.eye(NCLS, dtype=f32), jnp.full((4, 1), 1.0 / H, f32))

    hext1, self1 = pl.pallas_call(
        _prep1_body,
        grid=(_GRID,),
        in_specs=[_row_spec(D), _full_spec(D, HC1), _full_spec(HC1, H),
                  _full_spec(HC1, H), _full_spec(HC1, H), _full_spec(H, 16),
                  _full_spec(H, HC1)],
        out_specs=[_row_spec(ROWW), _row_spec(80)],
        out_shape=[jax.ShapeDtypeStruct((N, ROWW), f32),
                   jax.ShapeDtypeStruct((N, 80), f32)],
    )(x, w1cm, bdl1, bdr1, ob1, dup8, de8)

    p10, p11 = _edge_kernel_for(False, 1)(hext1, srcp, dstp)

    hext2, selfa, selfb = pl.pallas_call(
        _mid_body,
        grid=(_GRID,),
        in_specs=[_row_spec(ROWW), _row_spec(ROWW), _row_spec(80),
                  _full_spec(1, HC1), _full_spec(HC1, 64), _full_spec(HC1, 64),
                  _full_spec(64, 4), _full_spec(64, 4), _full_spec(64, 4),
                  _full_spec(64, 4), _full_spec(64, 4), _full_spec(4, 16),
                  _full_spec(4, 64), _full_spec(H, HC1)],
        out_specs=[_row_spec(2 * ROWW), _row_spec(80), _row_spec(80)],
        out_shape=[jax.ShapeDtypeStruct((N, 2 * ROWW), f32),
                   jax.ShapeDtypeStruct((N, 80), f32),
                   jax.ShapeDtypeStruct((N, 80), f32)],
    )(p10, p11, self1, b1[cm1].reshape(1, HC1), w2a, w2b,
      bdl2a, bdr2a, bdl2b, bdr2b, ob2, dup4, de4, de8)

    p20, p21 = _edge_kernel_for(True, 2)(hext2.reshape(2 * N, ROWW),
                                         srcp, dstp)

    out = pl.pallas_call(
        _fin_body,
        grid=(_GRID,),
        in_specs=[_row_spec(ROWW), _row_spec(ROWW), _row_spec(80),
                  _row_spec(80), _full_spec(1, NCLS), _full_spec(4, 64),
                  _full_spec(64, NCLS)],
        out_specs=[_row_spec(NCLS)],
        out_shape=[jax.ShapeDtypeStruct((N, NCLS), f32)],
    )(p20, p21, selfa, selfb, b2.reshape(1, NCLS), de4, meancm)[0]

    return out, jnp.float32(0.0)
